# baseline (speedup 1.0000x reference)
import jax
import jax.numpy as jnp
import numpy as np
from jax.experimental import pallas as pl
from jax.experimental.pallas import tpu as pltpu

ROW_BLK = 400
PANEL = 1280


def _sweep1_kernel(adj_ref, x_ref, w1_ref, w23_ref,
                   tw_ref, mv_ref, xw1_s, tw_s):
    i = pl.program_id(0)

    @pl.when(i == 0)
    def _():
        xw1_s[...] = jnp.dot(
            x_ref[...], w1_ref[...], preferred_element_type=jnp.float32)
        tw_s[...] = jnp.zeros_like(tw_s)

    mv_ref[...] = jnp.dot(adj_ref[...], tw_s[...],
                          preferred_element_type=jnp.float32)

    temp = jnp.maximum(
        jnp.dot(adj_ref[...], xw1_s[...],
                preferred_element_type=jnp.float32), 0.0)
    twi = jnp.dot(temp, w23_ref[...], preferred_element_type=jnp.float32)
    tw_s[pl.ds(i * ROW_BLK, ROW_BLK), :] = twi
    tw_ref[...] = twi


def _decoder_kernel(z_ref, zt_ref, out_ref):
    out_ref[...] = jnp.dot(z_ref[...], zt_ref[...],
                           preferred_element_type=jnp.float32)


def kernel(node_vectors, adj, W1, W2, W3):
    n, d = node_vectors.shape
    h1 = W1.shape[1]
    h2 = W2.shape[1]
    w23 = jnp.concatenate([W2, W3], axis=1)

    nblk = n // ROW_BLK
    npan = -(-n // PANEL)

    tw, mv = pl.pallas_call(
        _sweep1_kernel,
        grid=(nblk,),
        in_specs=[
            pl.BlockSpec((ROW_BLK, n), lambda i: (i, 0)),
            pl.BlockSpec((n, d), lambda i: (0, 0)),
            pl.BlockSpec((d, h1), lambda i: (0, 0)),
            pl.BlockSpec((h1, 2 * h2), lambda i: (0, 0)),
        ],
        out_specs=[
            pl.BlockSpec((ROW_BLK, 2 * h2), lambda i: (i, 0)),
            pl.BlockSpec((ROW_BLK, 2 * h2), lambda i: (i, 0)),
        ],
        out_shape=[
            jax.ShapeDtypeStruct((n, 2 * h2), jnp.float32),
            jax.ShapeDtypeStruct((n, 2 * h2), jnp.float32),
        ],
        scratch_shapes=[
            pltpu.VMEM((n, h1), jnp.float32),
            pltpu.VMEM((n, 2 * h2), jnp.float32),
        ],
    )(adj, node_vectors, W1, w23)

    si_l, sp_l = [], []
    for i in range(nblk):
        for p in range(i * ROW_BLK // PANEL, npan):
            si_l.append(i)
            sp_l.append(p)
    si_a = jnp.asarray(np.asarray(si_l, dtype=np.int32))
    sp_a = jnp.asarray(np.asarray(sp_l, dtype=np.int32))
    nsteps = len(si_l)

    def _sweep2_kernel(si_ref, sp_ref, adj_ref, tw_ref, mv_ref,
                       mean_ref, logvar_ref, acc):
        t = pl.program_id(0)
        i = si_ref[t]
        p = sp_ref[t]

        rows = jax.lax.broadcasted_iota(jnp.int32, tw_ref.shape, 0) + p * PANEL
        twm = jnp.where((rows >= i * ROW_BLK) & (rows < n), tw_ref[...], 0.0)

        def _update(contrib):
            first = p == (i * ROW_BLK) // PANEL
            acc[...] = jnp.where(first, mv_ref[...], acc[...]) + contrib

        @pl.when(p < npan - 1)
        def _():
            _update(jnp.dot(adj_ref[...], twm,
                            preferred_element_type=jnp.float32))

        @pl.when(p == npan - 1)
        def _():
            cols = jax.lax.broadcasted_iota(jnp.int32, adj_ref.shape, 1)
            adjm = jnp.where(cols < n - p * PANEL, adj_ref[...], 0.0)
            _update(jnp.dot(adjm, twm, preferred_element_type=jnp.float32))
            mean_ref[...] = acc[:, :h2]
            logvar_ref[...] = acc[:, h2:]

    grid_spec = pltpu.PrefetchScalarGridSpec(
        num_scalar_prefetch=2,
        grid=(nsteps,),
        in_specs=[
            pl.BlockSpec((ROW_BLK, PANEL),
                         lambda t, si, sp: (si[t], sp[t])),
            pl.BlockSpec((PANEL, 2 * h2),
                         lambda t, si, sp: (sp[t], 0)),
            pl.BlockSpec((ROW_BLK, 2 * h2),
                         lambda t, si, sp: (si[t], 0)),
        ],
        out_specs=[
            pl.BlockSpec((ROW_BLK, h2),
                         lambda t, si, sp: (si[t], 0)),
            pl.BlockSpec((ROW_BLK, h2),
                         lambda t, si, sp: (si[t], 0)),
        ],
        scratch_shapes=[pltpu.VMEM((ROW_BLK, 2 * h2), jnp.float32)],
    )
    mean, logvar = pl.pallas_call(
        _sweep2_kernel,
        grid_spec=grid_spec,
        out_shape=[
            jax.ShapeDtypeStruct((n, h2), jnp.float32),
            jax.ShapeDtypeStruct((n, h2), jnp.float32),
        ],
    )(si_a, sp_a, adj, tw, mv)

    adj_dec = pl.pallas_call(
        _decoder_kernel,
        grid=(nblk,),
        in_specs=[
            pl.BlockSpec((ROW_BLK, h2), lambda i: (i, 0)),
            pl.BlockSpec((h2, n), lambda i: (0, 0)),
        ],
        out_specs=pl.BlockSpec((ROW_BLK, n), lambda i: (i, 0)),
        out_shape=jax.ShapeDtypeStruct((n, n), jnp.float32),
    )(mean, mean.T)

    return (adj_dec, mean, logvar)


# speedup vs baseline: 1.0844x; 1.0844x over previous
import jax
import jax.numpy as jnp
import numpy as np
from jax.experimental import pallas as pl
from jax.experimental.pallas import tpu as pltpu

ROW_BLK = 400
PANEL = 5120


def _sweep1_kernel(adj_ref, x_ref, w1_ref, w23_ref,
                   tw_ref, mv_ref, xw1_s, tw_s):
    i = pl.program_id(0)

    @pl.when(i == 0)
    def _():
        xw1_s[...] = jnp.dot(
            x_ref[...], w1_ref[...], preferred_element_type=jnp.float32)
        tw_s[...] = jnp.zeros_like(tw_s)

    mv_ref[...] = jnp.dot(adj_ref[...], tw_s[...],
                          preferred_element_type=jnp.float32)

    temp = jnp.maximum(
        jnp.dot(adj_ref[...], xw1_s[...],
                preferred_element_type=jnp.float32), 0.0)
    twi = jnp.dot(temp, w23_ref[...], preferred_element_type=jnp.float32)
    tw_s[pl.ds(i * ROW_BLK, ROW_BLK), :] = twi
    tw_ref[...] = twi


def _decoder_kernel(z_ref, zt_ref, out_ref):
    out_ref[...] = jnp.dot(z_ref[...], zt_ref[...],
                           preferred_element_type=jnp.float32)


def kernel(node_vectors, adj, W1, W2, W3):
    n, d = node_vectors.shape
    h1 = W1.shape[1]
    h2 = W2.shape[1]
    w23 = jnp.concatenate([W2, W3], axis=1)

    nblk = n // ROW_BLK
    npan = -(-n // PANEL)

    tw, mv = pl.pallas_call(
        _sweep1_kernel,
        grid=(nblk,),
        in_specs=[
            pl.BlockSpec((ROW_BLK, n), lambda i: (i, 0)),
            pl.BlockSpec((n, d), lambda i: (0, 0)),
            pl.BlockSpec((d, h1), lambda i: (0, 0)),
            pl.BlockSpec((h1, 2 * h2), lambda i: (0, 0)),
        ],
        out_specs=[
            pl.BlockSpec((ROW_BLK, 2 * h2), lambda i: (i, 0)),
            pl.BlockSpec((ROW_BLK, 2 * h2), lambda i: (i, 0)),
        ],
        out_shape=[
            jax.ShapeDtypeStruct((n, 2 * h2), jnp.float32),
            jax.ShapeDtypeStruct((n, 2 * h2), jnp.float32),
        ],
        scratch_shapes=[
            pltpu.VMEM((n, h1), jnp.float32),
            pltpu.VMEM((n, 2 * h2), jnp.float32),
        ],
    )(adj, node_vectors, W1, w23)

    si_l, sp_l = [], []
    for i in range(nblk):
        for p in range(i * ROW_BLK // PANEL, npan):
            si_l.append(i)
            sp_l.append(p)
    si_a = jnp.asarray(np.asarray(si_l, dtype=np.int32))
    sp_a = jnp.asarray(np.asarray(sp_l, dtype=np.int32))
    nsteps = len(si_l)

    def _sweep2_kernel(si_ref, sp_ref, adj_ref, tw_ref, mv_ref,
                       mean_ref, logvar_ref, acc):
        t = pl.program_id(0)
        i = si_ref[t]
        p = sp_ref[t]

        rows = jax.lax.broadcasted_iota(jnp.int32, tw_ref.shape, 0) + p * PANEL
        twm = jnp.where((rows >= i * ROW_BLK) & (rows < n), tw_ref[...], 0.0)

        def _update(contrib):
            first = p == (i * ROW_BLK) // PANEL
            acc[...] = jnp.where(first, mv_ref[...], acc[...]) + contrib

        @pl.when(p < npan - 1)
        def _():
            _update(jnp.dot(adj_ref[...], twm,
                            preferred_element_type=jnp.float32))

        @pl.when(p == npan - 1)
        def _():
            cols = jax.lax.broadcasted_iota(jnp.int32, adj_ref.shape, 1)
            adjm = jnp.where(cols < n - p * PANEL, adj_ref[...], 0.0)
            _update(jnp.dot(adjm, twm, preferred_element_type=jnp.float32))
            mean_ref[...] = acc[:, :h2]
            logvar_ref[...] = acc[:, h2:]

    grid_spec = pltpu.PrefetchScalarGridSpec(
        num_scalar_prefetch=2,
        grid=(nsteps,),
        in_specs=[
            pl.BlockSpec((ROW_BLK, PANEL),
                         lambda t, si, sp: (si[t], sp[t])),
            pl.BlockSpec((PANEL, 2 * h2),
                         lambda t, si, sp: (sp[t], 0)),
            pl.BlockSpec((ROW_BLK, 2 * h2),
                         lambda t, si, sp: (si[t], 0)),
        ],
        out_specs=[
            pl.BlockSpec((ROW_BLK, h2),
                         lambda t, si, sp: (si[t], 0)),
            pl.BlockSpec((ROW_BLK, h2),
                         lambda t, si, sp: (si[t], 0)),
        ],
        scratch_shapes=[pltpu.VMEM((ROW_BLK, 2 * h2), jnp.float32)],
    )
    mean, logvar = pl.pallas_call(
        _sweep2_kernel,
        grid_spec=grid_spec,
        out_shape=[
            jax.ShapeDtypeStruct((n, h2), jnp.float32),
            jax.ShapeDtypeStruct((n, h2), jnp.float32),
        ],
    )(si_a, sp_a, adj, tw, mv)

    adj_dec = pl.pallas_call(
        _decoder_kernel,
        grid=(nblk,),
        in_specs=[
            pl.BlockSpec((ROW_BLK, h2), lambda i: (i, 0)),
            pl.BlockSpec((h2, n), lambda i: (0, 0)),
        ],
        out_specs=pl.BlockSpec((ROW_BLK, n), lambda i: (i, 0)),
        out_shape=jax.ShapeDtypeStruct((n, n), jnp.float32),
    )(mean, mean.T)

    return (adj_dec, mean, logvar)


# speedup vs baseline: 1.1294x; 1.0415x over previous
import jax
import jax.numpy as jnp
from jax.experimental import pallas as pl
from jax.experimental.pallas import tpu as pltpu

ROW_BLK = 400


def _gcn_kernel(adj_ref, x_ref, w1_ref, w23_ref,
                mean_ref, logvar_ref,
                xw1_s, tw_s):
    p = pl.program_id(0)
    i = pl.program_id(1)
    h2 = mean_ref.shape[1]

    @pl.when((p == 0) & (i == 0))
    def _():
        xw1_s[...] = jnp.dot(
            x_ref[...], w1_ref[...], preferred_element_type=jnp.float32)

    @pl.when(p == 0)
    def _():
        temp = jnp.maximum(
            jnp.dot(adj_ref[...], xw1_s[...],
                    preferred_element_type=jnp.float32), 0.0)
        tw_s[pl.ds(i * ROW_BLK, ROW_BLK), :] = jnp.dot(
            temp, w23_ref[...], preferred_element_type=jnp.float32)

    @pl.when(p == 1)
    def _():
        mv = jnp.dot(adj_ref[...], tw_s[...],
                     preferred_element_type=jnp.float32)
        mean_ref[...] = mv[:, :h2]
        logvar_ref[...] = mv[:, h2:]


def _decoder_kernel(z_ref, zt_ref, out_ref):
    out_ref[...] = jnp.dot(z_ref[...], zt_ref[...],
                           preferred_element_type=jnp.float32)


def kernel(node_vectors, adj, W1, W2, W3):
    n, d = node_vectors.shape
    h1 = W1.shape[1]
    h2 = W2.shape[1]
    w23 = jnp.concatenate([W2, W3], axis=1)

    nblk = n // ROW_BLK
    last = nblk - 1

    mean, logvar = pl.pallas_call(
        _gcn_kernel,
        grid=(2, nblk),
        in_specs=[
            pl.BlockSpec((ROW_BLK, n), lambda p, i: (i, 0)),
            pl.BlockSpec((n, d), lambda p, i: (0, 0)),
            pl.BlockSpec((d, h1), lambda p, i: (0, 0)),
            pl.BlockSpec((h1, 2 * h2), lambda p, i: (0, 0)),
        ],
        out_specs=[
            pl.BlockSpec((ROW_BLK, h2),
                         lambda p, i: (jnp.where(p == 1, i, 0), 0)),
            pl.BlockSpec((ROW_BLK, h2),
                         lambda p, i: (jnp.where(p == 1, i, 0), 0)),
        ],
        out_shape=[
            jax.ShapeDtypeStruct((n, h2), jnp.float32),
            jax.ShapeDtypeStruct((n, h2), jnp.float32),
        ],
        scratch_shapes=[
            pltpu.VMEM((n, h1), jnp.float32),
            pltpu.VMEM((n, 2 * h2), jnp.float32),
        ],
        compiler_params=pltpu.CompilerParams(
            dimension_semantics=("arbitrary", "arbitrary")),
    )(adj, node_vectors, W1, w23)

    adj_dec = pl.pallas_call(
        _decoder_kernel,
        grid=(nblk,),
        in_specs=[
            pl.BlockSpec((ROW_BLK, h2), lambda i: (i, 0)),
            pl.BlockSpec((h2, n), lambda i: (0, 0)),
        ],
        out_specs=pl.BlockSpec((ROW_BLK, n), lambda i: (i, 0)),
        out_shape=jax.ShapeDtypeStruct((n, n), jnp.float32),
    )(mean, mean.T)

    return (adj_dec, mean, logvar)
